# baseline (speedup 1.0000x reference)
import jax
import jax.numpy as jnp
from jax import lax
from jax.experimental import pallas as pl
from jax.experimental.pallas import tpu as pltpu
from jax.experimental.pallas import tpu_sc as plsc

C = 0.01
SQRT_C = C ** 0.5

N = 10000
D = 128
HD = D // 2
E = 320000
R = 200

NC = 2
NS = 16
EPT = E // NS
B = 80
S = 5
SB = S * B
NSUPER = EPT // SB
NPAD = 10240
RPT = NPAD // NS
ROW_BLK = 2000


def _tc1_body(x_ref, th_ref, th2_ref, r_ref):
    x = x_ref[...]
    xn = jnp.sqrt(jnp.sum(x * x, axis=1, keepdims=True))
    xnc = jnp.maximum(xn, 1e-10)
    s = jnp.minimum(SQRT_C * xnc, 1.0 - 1e-5)
    at = 0.5 * jnp.log((1.0 + s) / (1.0 - s))
    th = x * (at / (SQRT_C * xnc))
    th_ref[...] = th
    thb = th.astype(jnp.bfloat16)
    th2_ref[0] = thb[:, :HD]
    th2_ref[1] = thb[:, HD:]
    r_ref[...] = (2.0 / SQRT_C) * at


_tc1 = pl.pallas_call(
    _tc1_body,
    grid=(N // ROW_BLK,),
    in_specs=[pl.BlockSpec((ROW_BLK, D), lambda i: (i, 0))],
    out_specs=[
        pl.BlockSpec((ROW_BLK, D), lambda i: (i, 0)),
        pl.BlockSpec((2, ROW_BLK, HD), lambda i: (0, i, 0)),
        pl.BlockSpec((ROW_BLK, 1), lambda i: (i, 0)),
    ],
    out_shape=[
        jax.ShapeDtypeStruct((N, D), jnp.float32),
        jax.ShapeDtypeStruct((2, N, HD), jnp.bfloat16),
        jax.ShapeDtypeStruct((N, 1), jnp.float32),
    ],
)


def _tc2_body(accl_ref, accr_ref, deg_ref, th_ref, nrm_ref, wn_ref, wl_ref,
              we_ref, o_ref):
    acc = jnp.concatenate((accl_ref[...], accr_ref[...]), axis=1)
    deg = deg_ref[...][:, :1]
    th = th_ref[...]
    h1 = jnp.dot(acc, wn_ref[...], preferred_element_type=jnp.float32)
    h1 = jnp.clip(h1 * nrm_ref[...], -10.0, 10.0)
    lm = jnp.where(
        deg > 0.5,
        jnp.dot(th, wl_ref[...], preferred_element_type=jnp.float32),
        jnp.dot(th, we_ref[...], preferred_element_type=jnp.float32),
    )
    h2 = jnp.clip(h1 + lm, -10.0, 10.0)
    vn = jnp.maximum(jnp.sqrt(jnp.sum(h2 * h2, axis=1, keepdims=True)), 1e-10)
    o_ref[...] = jnp.tanh(SQRT_C * vn) * (h2 / (SQRT_C * vn))


_tc2 = pl.pallas_call(
    _tc2_body,
    grid=(N // ROW_BLK,),
    in_specs=[
        pl.BlockSpec((ROW_BLK, HD), lambda i: (i, 0)),
        pl.BlockSpec((ROW_BLK, HD), lambda i: (i, 0)),
        pl.BlockSpec((ROW_BLK, 16), lambda i: (i, 0)),
        pl.BlockSpec((ROW_BLK, D), lambda i: (i, 0)),
        pl.BlockSpec((ROW_BLK, 1), lambda i: (i, 0)),
        pl.BlockSpec((D, D), lambda i: (0, 0)),
        pl.BlockSpec((D, D), lambda i: (0, 0)),
        pl.BlockSpec((D, D), lambda i: (0, 0)),
    ],
    out_specs=pl.BlockSpec((ROW_BLK, D), lambda i: (i, 0)),
    out_shape=jax.ShapeDtypeStruct((N, D), jnp.float32),
)


def _sc_body(th2_hbm, rad_hbm, src_hbm, dst_hbm, et_hbm, relp_hbm, z64_hbm,
             z16_hbm, acc_out, deg_out, sraw_v, draw_v, eraw_v, srcg2_v,
             dstc2_v, ets_v, rw_v, h_rows, out_rows, ones_rows, rel_v, rad_v,
             isem, gsem, ssem, dsem, acc_sh, deg_sh):
    c = lax.axis_index("c")
    s = lax.axis_index("s")
    rows0 = s * RPT

    pltpu.sync_copy(z64_hbm.at[pl.ds(rows0, RPT)], acc_sh.at[pl.ds(rows0, RPT)])
    pltpu.sync_copy(z16_hbm.at[pl.ds(rows0, RPT)], deg_sh.at[pl.ds(rows0, RPT)])
    pltpu.sync_copy(relp_hbm.at[pl.ds(c * (R * HD // 2), R * HD // 2)], rel_v)
    pltpu.sync_copy(rad_hbm, rad_v)

    iota = lax.iota(jnp.int32, 16)
    onehot = jnp.where(iota == 0, 1.0, 0.0).astype(jnp.float32)

    def fill_ones(i, carry):
        ones_rows[i, :] = onehot
        return carry

    lax.fori_loop(0, B, fill_ones, 0)
    plsc.subcore_barrier()

    coff = c * N
    e0 = s * EPT

    def issue_idx(g, b):
        base = e0 + g * SB
        pltpu.async_copy(src_hbm.at[pl.ds(base, SB)], sraw_v[b], isem[b])
        pltpu.async_copy(dst_hbm.at[pl.ds(base, SB)], draw_v[b], isem[b])
        pltpu.async_copy(et_hbm.at[pl.ds(base, SB)], eraw_v[b], isem[b])

    def wait_idx(b):
        pltpu.make_async_copy(src_hbm.at[pl.ds(0, SB)], sraw_v[b],
                              isem[b]).wait()
        pltpu.make_async_copy(dst_hbm.at[pl.ds(0, SB)], draw_v[b],
                              isem[b]).wait()
        pltpu.make_async_copy(et_hbm.at[pl.ds(0, SB)], eraw_v[b],
                              isem[b]).wait()

    def prep(b):
        for k in range(SB // 16):
            sl = pl.ds(k * 16, 16)
            j, r16 = divmod(k * 16, B)
            sl2 = pl.ds(r16, 16)
            s16 = sraw_v[b][sl]
            d16 = draw_v[b][sl]
            srcg2_v[b][j, sl2] = s16 + coff
            dstc2_v[b][j, sl2] = d16
            ets_v[b][sl] = eraw_v[b][sl] * (HD // 2)
            rs = plsc.load_gather(rad_v, [s16])
            rd = plsc.load_gather(rad_v, [d16])
            rw_v[b][sl] = jnp.exp(-jnp.abs(rs - rd))

    def issue_gathers(b):
        for j in range(S):
            pltpu.async_copy(th2_hbm.at[srcg2_v[b].at[j]],
                             h_rows[b].at[pl.ds(j * B, B)], gsem[b])

    def wait_gathers(b):
        for j in range(S):
            pltpu.make_async_copy(th2_hbm.at[srcg2_v[b].at[j]],
                                  h_rows[b].at[pl.ds(j * B, B)],
                                  gsem[b]).wait()

    def issue_scatters(b):
        for j in range(S):
            pltpu.async_copy(out_rows.at[pl.ds(j * B, B)],
                             acc_sh.at[dstc2_v[b].at[j]], ssem, add=True)

        @pl.when(c == 0)
        def _():
            for j in range(S):
                pltpu.async_copy(ones_rows, deg_sh.at[dstc2_v[b].at[j]],
                                 dsem, add=True)

    def drain_scatters(b):
        for j in range(S):
            pltpu.make_async_copy(out_rows.at[pl.ds(j * B, B)],
                                  acc_sh.at[dstc2_v[b].at[j]], ssem).wait()

        @pl.when(c == 0)
        def _():
            for j in range(S):
                pltpu.make_async_copy(ones_rows, deg_sh.at[dstc2_v[b].at[j]],
                                      dsem).wait()

    def edge_pass(b):
        @plsc.parallel_loop(0, SB, unroll=8)
        def _(e):
            eb = lax.broadcast(e, (16,))
            rw = plsc.load_gather(rw_v[b], [eb])
            rbase = plsc.load_gather(ets_v[b], [eb])
            for g in range(HD // 32):
                hv2 = h_rows[b][e, pl.ds(g * 32, 32)]
                ha, hb = plsc.unpack(hv2, format=plsc.PackFormat.INTERLEAVED)
                rp = plsc.load_gather(rel_v, [rbase + g * 16 + iota])
                rv2 = plsc.bitcast(rp, jnp.bfloat16)
                ra, rb = plsc.unpack(rv2, format=plsc.PackFormat.INTERLEAVED)
                out_rows[e, pl.ds(g * 32, 16)] = rw * (ha + ra)
                out_rows[e, pl.ds(g * 32 + 16, 16)] = rw * (hb + rb)

    issue_idx(0, 0)
    wait_idx(0)
    prep(0)
    issue_gathers(0)
    issue_idx(1, 1)

    def outer(g, carry):
        for b in (0, 1):
            t = g * 2 + b
            nb = 1 - b

            @pl.when(t >= 1)
            def _():
                drain_scatters(nb)

            @pl.when(t + 1 < NSUPER)
            def _():
                wait_idx(nb)
                prep(nb)

            @pl.when(t + 2 < NSUPER)
            def _():
                issue_idx(t + 2, b)

            @pl.when(t + 1 < NSUPER)
            def _():
                issue_gathers(nb)

            wait_gathers(b)
            edge_pass(b)
            issue_scatters(b)

        return carry

    lax.fori_loop(0, NSUPER // 2, outer, 0)

    drain_scatters(1)
    plsc.subcore_barrier()

    pltpu.sync_copy(acc_sh.at[pl.ds(rows0, RPT)],
                    acc_out.at[pl.ds(c * NPAD + rows0, RPT)])

    @pl.when(c == 0)
    def _():
        pltpu.sync_copy(deg_sh.at[pl.ds(rows0, RPT)],
                        deg_out.at[pl.ds(rows0, RPT)])


_sc = pl.kernel(
    _sc_body,
    mesh=plsc.VectorSubcoreMesh(core_axis_name="c", subcore_axis_name="s"),
    compiler_params=pltpu.CompilerParams(
        needs_layout_passes=False, use_tc_tiling_on_sc=False),
    out_type=[
        jax.ShapeDtypeStruct((NC * NPAD, HD), jnp.float32),
        jax.ShapeDtypeStruct((NPAD, 16), jnp.float32),
    ],
    scratch_types=[
        [pltpu.VMEM((SB,), jnp.int32) for _ in range(2)],
        [pltpu.VMEM((SB,), jnp.int32) for _ in range(2)],
        [pltpu.VMEM((SB,), jnp.int32) for _ in range(2)],
        [pltpu.VMEM((S, B), jnp.int32) for _ in range(2)],
        [pltpu.VMEM((S, B), jnp.int32) for _ in range(2)],
        [pltpu.VMEM((SB,), jnp.int32) for _ in range(2)],
        [pltpu.VMEM((SB,), jnp.float32) for _ in range(2)],
        [pltpu.VMEM((SB, HD), jnp.bfloat16) for _ in range(2)],
        pltpu.VMEM((SB, HD), jnp.float32),
        pltpu.VMEM((B, 16), jnp.float32),
        pltpu.VMEM((R * HD // 2,), jnp.int32),
        pltpu.VMEM((N,), jnp.float32),
        [pltpu.SemaphoreType.DMA for _ in range(2)],
        [pltpu.SemaphoreType.DMA for _ in range(2)],
        pltpu.SemaphoreType.DMA,
        pltpu.SemaphoreType.DMA,
        pltpu.VMEM_SHARED((NPAD, HD), jnp.float32),
        pltpu.VMEM_SHARED((NPAD, 16), jnp.float32),
    ],
)


def kernel(h_hyper, edge_index, edge_type, rel_emb, norm, weight_neighbor,
           loop_weight, evolve_loop_weight):
    src = edge_index[0]
    dst = edge_index[1]
    th, th2, rad = _tc1(h_hyper)
    rad = rad.reshape(N)
    th2 = th2.reshape(2, N, HD // 32, 2, 16).transpose(0, 1, 2, 4, 3)
    th2 = th2.reshape(2 * N, HD)
    relb = rel_emb.astype(jnp.bfloat16)
    relb = jnp.stack((relb[:, :HD], relb[:, HD:]))
    relb = relb.reshape(2, R, HD // 32, 2, 16).transpose(0, 1, 2, 4, 3)
    relp = jax.lax.bitcast_convert_type(
        relb.reshape(2, R, HD // 32, 16, 2), jnp.int32)
    relp = relp.reshape(NC * R * HD // 2)
    z64 = jnp.zeros((NPAD, HD), jnp.float32)
    z16 = jnp.zeros((NPAD, 16), jnp.float32)
    acc2, deg = _sc(th2, rad, src, dst, edge_type, relp, z64, z16)
    accl = acc2[:N]
    accr = acc2[NPAD:NPAD + N]
    return _tc2(accl, accr, deg[:N], th, norm, weight_neighbor, loop_weight,
                evolve_loop_weight)
